# baseline (speedup 1.0000x reference)
import functools

import jax
import jax.numpy as jnp
from jax import lax
from jax.experimental import pallas as pl
from jax.experimental.pallas import tpu as pltpu
from jax.experimental.pallas import tpu_sc as plsc

_VOCAB = 30522
_D = 768
_B = 64
_S = 512
_EPS = 1e-12
_L = 16
_NW = 32
_SP = _S // _NW
_NCH = _D // _L
_NPR = _NCH // 2
_NBUF = 4
_NOUT = 2


def _lane_perm(vec, idx):
    return lax.gather(
        vec, idx[:, None],
        lax.GatherDimensionNumbers(offset_dims=(), collapsed_slice_dims=(0,),
                                   start_index_map=(0,)),
        slice_sizes=(1,), mode=lax.GatherScatterMode.PROMISE_IN_BOUNDS)


def _all_sum(v):
    for sh in (8, 4, 2, 1):
        idx = jnp.arange(_L, dtype=jnp.int32) ^ sh
        v = v + _lane_perm(v, idx)
    return v


def _rsqrt_newton(xv):
    iv = lax.bitcast_convert_type(xv, jnp.int32)
    iv = jnp.int32(0x5F3759DF) - lax.shift_right_logical(iv, 1)
    yv = lax.bitcast_convert_type(iv, jnp.float32)
    for _ in range(3):
        yv = yv * (1.5 - 0.5 * xv * yv * yv)
    return yv


def _pack2(a, b):
    ba = lax.bitcast_convert_type(a, jnp.int32)
    bb = lax.bitcast_convert_type(b, jnp.int32)
    ra = ba + jnp.int32(0x7FFF) + (lax.shift_right_logical(ba, 16) & 1)
    rb = bb + jnp.int32(0x7FFF) + (lax.shift_right_logical(bb, 16) & 1)
    return lax.shift_right_logical(ra, 16) | (rb & jnp.int32(-65536))


def _unpack2(w):
    a = lax.bitcast_convert_type(lax.shift_left(w, 16), jnp.float32)
    b = lax.bitcast_convert_type(w & jnp.int32(-65536), jnp.float32)
    return a, b


def _sc_embed_ln(ids_hbm, tt_hbm, word_hbm, ttemb_hbm, pos_hbm, w_hbm, b_hbm,
                 out_hbm, comb_hbm, idsbuf, ttslab, idxc, inbuf, outbuf, pt,
                 tt2, wt, bs, wtbf, bsbf, wsem, csem, osem):
    wid = lax.axis_index("s") * 2 + lax.axis_index("c")
    s0 = pl.multiple_of(wid * _SP, _SP)
    cb = pl.multiple_of(wid * 2 * _SP, 2 * _SP)

    def _slab_fire(b, _):
        pltpu.async_copy(ids_hbm.at[b, pl.ds(s0, _SP)], idsbuf.at[b],
                         wsem.at[0])
        pltpu.async_copy(tt_hbm.at[b, pl.ds(s0, _SP)], ttslab.at[b],
                         wsem.at[1])
        return 0

    def _slab_drain(b, _):
        pltpu.make_async_copy(ids_hbm.at[b, pl.ds(s0, _SP)], idsbuf.at[b],
                              wsem.at[0]).wait()
        pltpu.make_async_copy(tt_hbm.at[b, pl.ds(s0, _SP)], ttslab.at[b],
                              wsem.at[1]).wait()
        return 0

    lax.fori_loop(0, _B, _slab_fire, 0)
    lax.fori_loop(0, _B, _slab_drain, 0)

    pltpu.sync_copy(pos_hbm.at[pl.ds(s0, _SP)], pt)
    pltpu.sync_copy(ttemb_hbm, tt2)
    pltpu.sync_copy(w_hbm, wt)
    pltpu.sync_copy(b_hbm, bs)

    for j2 in range(_NPR):
        da = pl.ds(2 * j2 * _L, _L)
        db = pl.ds((2 * j2 + 1) * _L, _L)
        dp = pl.ds(j2 * _L, _L)
        wtbf[dp] = _pack2(wt[da], wt[db])
        bsbf[dp] = _pack2(bs[da], bs[db])

    def _comb_body(i, _):
        for j2 in range(_NPR):
            for d in (pl.ds(2 * j2 * _L, _L), pl.ds((2 * j2 + 1) * _L, _L)):
                pc = pt[i, d]
                outbuf[0, i, d] = pc + tt2[0, d]
                outbuf[1, i, d] = pc + tt2[1, d]
        return 0

    lax.fori_loop(0, _SP, _comb_body, 0)
    pltpu.sync_copy(outbuf.at[0], comb_hbm.at[pl.ds(cb, _SP)])
    pltpu.sync_copy(outbuf.at[1], comb_hbm.at[pl.ds(cb + _SP, _SP)])

    iota16 = jnp.arange(_L, dtype=jnp.int32)
    inv_d = jnp.float32(1.0 / _D)

    def _issue_word(b, k):
        pltpu.async_copy(word_hbm.at[idsbuf.at[b]], inbuf.at[k], wsem.at[k])

    def _wait_word(b, k):
        pltpu.make_async_copy(word_hbm.at[idsbuf.at[b]], inbuf.at[k],
                              wsem.at[k]).wait()

    def _issue_comb(b, k):
        idxc[k, ...] = cb + ttslab[b, ...] * _SP + iota16
        pltpu.async_copy(comb_hbm.at[idxc.at[k]], inbuf.at[k], csem.at[k],
                         add=True)

    def _wait_comb(b, k):
        pltpu.make_async_copy(comb_hbm.at[idxc.at[k]], inbuf.at[k],
                              csem.at[k]).wait()

    _issue_word(0, 0)
    _issue_word(1, 1)
    _wait_word(0, 0)
    _issue_comb(0, 0)

    def _compute_tile(b, k, ob):
        inb = inbuf.at[k]
        onb = outbuf.at[ob]

        def _tok_body(i, _):
            zero = jnp.zeros((_L,), jnp.float32)

            @plsc.parallel_loop(0, _NPR, unroll=8,
                                carry=(zero, zero, zero, zero))
            def _pass1(j2, accs):
                a0, a1, q0, q1 = accs
                base = pl.multiple_of(j2 * 2 * _L, 2 * _L)
                ra = inb[i, pl.ds(base, _L)]
                rb = inb[i, pl.ds(base + _L, _L)]
                return (a0 + ra, a1 + rb, q0 + ra * ra, q1 + rb * rb)

            a0, a1, q0, q1 = _pass1
            mean_v = _all_sum(a0 + a1) * inv_d
            var_v = _all_sum(q0 + q1) * inv_d - mean_v * mean_v
            inv_v = _rsqrt_newton(var_v + jnp.float32(_EPS))

            @plsc.parallel_loop(0, _NPR, unroll=8)
            def _pass2(j2):
                base = pl.multiple_of(j2 * 2 * _L, 2 * _L)
                da = pl.ds(base, _L)
                db = pl.ds(base + _L, _L)
                dp = pl.ds(pl.multiple_of(j2 * _L, _L), _L)
                wa, wb2 = _unpack2(wtbf[dp])
                ba, bb = _unpack2(bsbf[dp])
                onb[i, da] = (inb[i, da] - mean_v) * inv_v * wa + ba
                onb[i, db] = (inb[i, db] - mean_v) * inv_v * wb2 + bb

            return 0

        lax.fori_loop(0, _SP, _tok_body, 0)

    def _outer_body(g, _):
        for k in range(_NBUF):
            ob = k % _NOUT
            b = g * _NBUF + k
            _wait_comb(b, k)

            @pl.when(b >= _NOUT)
            def _():
                pltpu.make_async_copy(
                    outbuf.at[ob], out_hbm.at[b - _NOUT, pl.ds(s0, _SP)],
                    osem.at[ob]).wait()

            _compute_tile(b, k, ob)

            @pl.when(b + 2 < _B)
            def _():
                _issue_word(b + 2, (k + 2) % _NBUF)

            @pl.when(b + 1 < _B)
            def _():
                _wait_word(b + 1, (k + 1) % _NBUF)
                _issue_comb(b + 1, (k + 1) % _NBUF)

            pltpu.async_copy(outbuf.at[ob], out_hbm.at[b, pl.ds(s0, _SP)],
                             osem.at[ob])
        return 0

    lax.fori_loop(0, _B // _NBUF, _outer_body, 0)

    for t in range(_NOUT):
        b = _B - _NOUT + t
        pltpu.make_async_copy(
            outbuf.at[b % _NOUT], out_hbm.at[b, pl.ds(s0, _SP)],
            osem.at[b % _NOUT]).wait()


def kernel(input_ids, attention_mask, token_type_ids, word_emb,
           token_type_emb, position_emb, ln_weight, ln_bias):
    del attention_mask
    mesh = plsc.VectorSubcoreMesh(core_axis_name="c", subcore_axis_name="s")
    k = functools.partial(
        pl.kernel,
        mesh=mesh,
        compiler_params=pltpu.CompilerParams(needs_layout_passes=False),
        out_type=(
            jax.ShapeDtypeStruct((_B, _S, _D), jnp.float32),
            jax.ShapeDtypeStruct((_NW * 2 * _SP, _D), jnp.float32),
        ),
        scratch_types=[
            pltpu.VMEM((_B, _SP), jnp.int32),
            pltpu.VMEM((_B, _SP), jnp.int32),
            pltpu.VMEM((_NBUF, _SP), jnp.int32),
            pltpu.VMEM((_NBUF, _SP, _D), jnp.float32),
            pltpu.VMEM((_NOUT, _SP, _D), jnp.float32),
            pltpu.VMEM((_SP, _D), jnp.float32),
            pltpu.VMEM((2, _D), jnp.float32),
            pltpu.VMEM((_D,), jnp.float32),
            pltpu.VMEM((_D,), jnp.float32),
            pltpu.VMEM((_D // 2,), jnp.int32),
            pltpu.VMEM((_D // 2,), jnp.int32),
            pltpu.SemaphoreType.DMA((_NBUF,)),
            pltpu.SemaphoreType.DMA((_NBUF,)),
            pltpu.SemaphoreType.DMA((_NOUT,)),
        ],
    )(_sc_embed_ln)
    out, _ = k(input_ids, token_type_ids, word_emb, token_type_emb,
               position_emb, ln_weight, ln_bias)
    return out


# speedup vs baseline: 1.3380x; 1.3380x over previous
import functools

import jax
import jax.numpy as jnp
from jax import lax
from jax.experimental import pallas as pl
from jax.experimental.pallas import tpu as pltpu
from jax.experimental.pallas import tpu_sc as plsc

_VOCAB = 30522
_D = 768
_B = 64
_S = 512
_EPS = 1e-12
_L = 16
_NW = 32
_SP = _S // _NW
_NCH = _D // _L
_NPR = _NCH // 2
_NBUF = 4
_NOUT = 2


def _lane_perm(vec, idx):
    return lax.gather(
        vec, idx[:, None],
        lax.GatherDimensionNumbers(offset_dims=(), collapsed_slice_dims=(0,),
                                   start_index_map=(0,)),
        slice_sizes=(1,), mode=lax.GatherScatterMode.PROMISE_IN_BOUNDS)


def _lane_splat(vec, i):
    return _lane_perm(vec, jnp.full((_L,), i, dtype=jnp.int32))


def _all_sum(v):
    for sh in (8, 4, 2, 1):
        idx = jnp.arange(_L, dtype=jnp.int32) ^ sh
        v = v + _lane_perm(v, idx)
    return v


def _rsqrt_newton(xv):
    iv = lax.bitcast_convert_type(xv, jnp.int32)
    iv = jnp.int32(0x5F3759DF) - lax.shift_right_logical(iv, 1)
    yv = lax.bitcast_convert_type(iv, jnp.float32)
    for _ in range(2):
        yv = yv * (1.5 - 0.5 * xv * yv * yv)
    return yv


def _pack2(a, b):
    ba = lax.bitcast_convert_type(a, jnp.int32)
    bb = lax.bitcast_convert_type(b, jnp.int32)
    ra = ba + jnp.int32(0x7FFF) + (lax.shift_right_logical(ba, 16) & 1)
    rb = bb + jnp.int32(0x7FFF) + (lax.shift_right_logical(bb, 16) & 1)
    lo = lax.shift_right_logical(ra, 16)
    hi = rb & jnp.int32(-65536)
    return lo | hi


def _unpack2(w):
    a = lax.bitcast_convert_type(lax.shift_left(w, 16), jnp.float32)
    b = lax.bitcast_convert_type(w & jnp.int32(-65536), jnp.float32)
    return a, b


def _sc_embed_ln(ids_hbm, tt_hbm, word_hbm, ttemb_hbm, pos_hbm, w_hbm, b_hbm,
                 out_hbm, idsbuf, ttslab, inbuf, outbuf, pt, tt2, wt, bs,
                 cbf, wtbf, bsbf, gsem, osem):
    wid = lax.axis_index("s") * 2 + lax.axis_index("c")
    s0 = pl.multiple_of(wid * _SP, _SP)

    def _slab_fire(b, _):
        pltpu.async_copy(ids_hbm.at[b, pl.ds(s0, _SP)], idsbuf.at[b],
                         gsem.at[0])
        pltpu.async_copy(tt_hbm.at[b, pl.ds(s0, _SP)], ttslab.at[b],
                         gsem.at[1])
        return 0

    def _slab_drain(b, _):
        pltpu.make_async_copy(ids_hbm.at[b, pl.ds(s0, _SP)], idsbuf.at[b],
                              gsem.at[0]).wait()
        pltpu.make_async_copy(tt_hbm.at[b, pl.ds(s0, _SP)], ttslab.at[b],
                              gsem.at[1]).wait()
        return 0

    lax.fori_loop(0, _B, _slab_fire, 0)
    lax.fori_loop(0, _B, _slab_drain, 0)

    pltpu.sync_copy(pos_hbm.at[pl.ds(s0, _SP)], pt)
    pltpu.sync_copy(ttemb_hbm, tt2)
    pltpu.sync_copy(w_hbm, wt)
    pltpu.sync_copy(b_hbm, bs)

    for j2 in range(_NPR):
        da = pl.ds(2 * j2 * _L, _L)
        db = pl.ds((2 * j2 + 1) * _L, _L)
        dp = pl.ds(j2 * _L, _L)
        wtbf[dp] = _pack2(wt[da], wt[db])
        bsbf[dp] = _pack2(bs[da], bs[db])

    def _cbf_body(i, _):
        for j2 in range(_NPR):
            da = pl.ds(2 * j2 * _L, _L)
            db = pl.ds((2 * j2 + 1) * _L, _L)
            dp = pl.ds(j2 * _L, _L)
            cbf[i, dp] = _pack2(pt[i, da] + tt2[0, da],
                                pt[i, db] + tt2[0, db])
            cbf[i + _SP, dp] = _pack2(pt[i, da] + tt2[1, da],
                                      pt[i, db] + tt2[1, db])
        return 0

    lax.fori_loop(0, _SP, _cbf_body, 0)

    inv_d = jnp.float32(1.0 / _D)

    for k in range(_NBUF):
        pltpu.async_copy(word_hbm.at[idsbuf.at[k]], inbuf.at[k], gsem.at[k])

    def _compute_tile(b, k, ob):
        inb = inbuf.at[k]
        onb = outbuf.at[ob]
        ttv = ttslab[b, ...]
        iota16 = jnp.arange(_L, dtype=jnp.int32)

        def _tok_body(i, _):
            rowv = _lane_splat(ttv, i) * _SP + i
            zero = jnp.zeros((_L,), jnp.float32)

            @plsc.parallel_loop(0, _NPR, unroll=8,
                                carry=(zero, zero, zero, zero))
            def _pass1(j2, accs):
                a0, a1, q0, q1 = accs
                base = pl.multiple_of(j2 * 2 * _L, 2 * _L)
                da = pl.ds(base, _L)
                db = pl.ds(base + _L, _L)
                colv = iota16 + j2 * _L
                pa, pb = _unpack2(plsc.load_gather(cbf, [rowv, colv]))
                ra = inb[i, da] + pa
                rb = inb[i, db] + pb
                onb[i, da] = ra
                onb[i, db] = rb
                return (a0 + ra, a1 + rb, q0 + ra * ra, q1 + rb * rb)

            a0, a1, q0, q1 = _pass1
            mean_v = _all_sum(a0 + a1) * inv_d
            var_v = _all_sum(q0 + q1) * inv_d - mean_v * mean_v
            inv_v = _rsqrt_newton(var_v + jnp.float32(_EPS))

            @plsc.parallel_loop(0, _NPR, unroll=8)
            def _pass2(j2):
                base = pl.multiple_of(j2 * 2 * _L, 2 * _L)
                da = pl.ds(base, _L)
                db = pl.ds(base + _L, _L)
                dp = pl.ds(pl.multiple_of(j2 * _L, _L), _L)
                wa, wb2 = _unpack2(wtbf[dp])
                ba, bb = _unpack2(bsbf[dp])
                onb[i, da] = (onb[i, da] - mean_v) * inv_v * wa + ba
                onb[i, db] = (onb[i, db] - mean_v) * inv_v * wb2 + bb

            return 0

        lax.fori_loop(0, _SP, _tok_body, 0)

    def _outer_body(g, _):
        for k in range(_NBUF):
            ob = k % _NOUT
            b = g * _NBUF + k
            pltpu.make_async_copy(
                word_hbm.at[idsbuf.at[b]], inbuf.at[k], gsem.at[k]).wait()

            @pl.when(b >= _NOUT)
            def _():
                pltpu.make_async_copy(
                    outbuf.at[ob], out_hbm.at[b - _NOUT, pl.ds(s0, _SP)],
                    osem.at[ob]).wait()

            _compute_tile(b, k, ob)

            @pl.when(b + _NBUF < _B)
            def _():
                pltpu.async_copy(word_hbm.at[idsbuf.at[b + _NBUF]],
                                 inbuf.at[k], gsem.at[k])

            pltpu.async_copy(outbuf.at[ob], out_hbm.at[b, pl.ds(s0, _SP)],
                             osem.at[ob])
        return 0

    lax.fori_loop(0, _B // _NBUF, _outer_body, 0)

    for t in range(_NOUT):
        b = _B - _NOUT + t
        pltpu.make_async_copy(
            outbuf.at[b % _NOUT], out_hbm.at[b, pl.ds(s0, _SP)],
            osem.at[b % _NOUT]).wait()


def kernel(input_ids, attention_mask, token_type_ids, word_emb,
           token_type_emb, position_emb, ln_weight, ln_bias):
    del attention_mask
    mesh = plsc.VectorSubcoreMesh(core_axis_name="c", subcore_axis_name="s")
    k = functools.partial(
        pl.kernel,
        mesh=mesh,
        compiler_params=pltpu.CompilerParams(needs_layout_passes=False),
        out_type=jax.ShapeDtypeStruct((_B, _S, _D), jnp.float32),
        scratch_types=[
            pltpu.VMEM((_B, _SP), jnp.int32),
            pltpu.VMEM((_B, _SP), jnp.int32),
            pltpu.VMEM((_NBUF, _SP, _D), jnp.float32),
            pltpu.VMEM((_NOUT, _SP, _D), jnp.float32),
            pltpu.VMEM((_SP, _D), jnp.float32),
            pltpu.VMEM((2, _D), jnp.float32),
            pltpu.VMEM((_D,), jnp.float32),
            pltpu.VMEM((_D,), jnp.float32),
            pltpu.VMEM((2 * _SP, _D // 2), jnp.int32),
            pltpu.VMEM((_D // 2,), jnp.int32),
            pltpu.VMEM((_D // 2,), jnp.int32),
            pltpu.SemaphoreType.DMA((_NBUF,)),
            pltpu.SemaphoreType.DMA((_NOUT,)),
        ],
    )(_sc_embed_ln)
    return k(input_ids, token_type_ids, word_emb, token_type_emb,
             position_emb, ln_weight, ln_bias)
